# baseline (speedup 1.0000x reference)
import functools
import jax
import jax.numpy as jnp
from jax import lax
from jax.experimental import pallas as pl
from jax.experimental.pallas import tpu as pltpu
from jax.experimental.pallas import tpu_sc as plsc

_VOCAB = 50265
_H = 768
_PAD = 1
_EPS = 1e-12
_B, _S = 1024, 200

_L = 16
_NC, _NS = 2, 16
_NW = _NC * _NS
_TOK_W = _B * _S // _NW
_CH = 32
_NCHUNK = _TOK_W // _CH
_NBUF = 4
_TG = 8
_NJ = _H // _L


def _rsqrt16(v):
    i = plsc.bitcast(v, jnp.int32)
    i = jnp.int32(0x5F3759DF) - lax.shift_right_arithmetic(i, 1)
    y = plsc.bitcast(i, jnp.float32)
    for _ in range(3):
        y = y * (1.5 - 0.5 * v * y * y)
    return y


def _splat(s, dtype):
    return lax.broadcast_in_dim(s, (_L,), ()).astype(dtype)


def _emb_body(ids_hbm, word_hbm, pos_hbm, g_hbm, beta_hbm, out_hbm,
              ids_v, pos_v, wb, g_v, b_v, sems):
    wid = lax.axis_index("s") * _NC + lax.axis_index("c")
    base = wid * _TOK_W

    pltpu.sync_copy(g_hbm, g_v)
    pltpu.sync_copy(beta_hbm, b_v)
    pltpu.sync_copy(ids_hbm.at[pl.ds(base, _TOK_W)], ids_v)

    lane = lax.iota(jnp.int32, _L)
    low8 = (lane < 8).astype(jnp.int32)

    def std_chunk(off, carry):
        v = ids_v[pl.ds(off, _L)]
        m = (v != _PAD).astype(jnp.int32)
        pos = (lax.cumsum(m, axis=0) + carry) * m + _PAD
        pos_v[pl.ds(off, _L)] = pos
        return carry + _splat(jnp.sum(m), jnp.int32)

    def row_pair(p, _):
        pbase = p * (2 * _S)

        def chA(j, carry):
            return std_chunk(pbase + j * _L, carry)

        carry = lax.fori_loop(0, 12, chA, jnp.zeros((_L,), jnp.int32))

        off = pbase + 12 * _L
        v = ids_v[pl.ds(off, _L)]
        m = (v != _PAD).astype(jnp.int32)
        mlo = m * low8
        mhi = m - mlo
        poslo = (lax.cumsum(mlo, axis=0) + carry) * m + _PAD
        poshi = lax.cumsum(mhi, axis=0) * m + _PAD
        pos_v[pl.ds(off, _L)] = jnp.where(lane < 8, poslo, poshi)
        carry = _splat(jnp.sum(mhi), jnp.int32)

        def chB(j, carry):
            return std_chunk(pbase + 13 * _L + j * _L, carry)

        lax.fori_loop(0, 12, chB, carry)
        return 0

    lax.fori_loop(0, _TOK_W // (2 * _S), row_pair, 0)

    inv_h = 1.0 / _H

    def issue_gather_w(k, buf):
        idx_w = ids_v.at[pl.ds(k * _CH, _CH)]
        pltpu.async_copy(word_hbm.at[idx_w], wb.at[buf], sems.at[buf])

    def wait_gather_w(k, buf):
        pltpu.make_async_copy(word_hbm.at[ids_v.at[pl.ds(k * _CH, _CH)]],
                              wb.at[buf], sems.at[buf]).wait()

    def issue_gather_p(k, buf):
        idx_p = pos_v.at[pl.ds(k * _CH, _CH)]
        pltpu.async_copy(pos_hbm.at[idx_p], wb.at[buf], sems.at[_NBUF + buf],
                         add=True)

    def wait_gather_p(k, buf):
        pltpu.make_async_copy(pos_hbm.at[pos_v.at[pl.ds(k * _CH, _CH)]],
                              wb.at[buf], sems.at[_NBUF + buf]).wait()

    def drain_out(buf):
        pltpu.make_async_copy(wb.at[buf], out_hbm.at[pl.ds(base, _CH)],
                              sems.at[2 * _NBUF + buf]).wait()

    def compute_chunk(buf):
        def group(gidx, _):
            t0 = gidx * _TG

            def p1(j, acc):
                s, q = acc
                jo = j * _L
                xs = [wb[buf, t0 + tp, pl.ds(jo, _L)] for tp in range(_TG)]
                s = tuple(s[tp] + xs[tp] for tp in range(_TG))
                q = tuple(q[tp] + xs[tp] * xs[tp] for tp in range(_TG))
                return (s, q)

            zeros = tuple(jnp.zeros((_L,), jnp.float32) for _ in range(_TG))
            s, q = plsc.parallel_loop(0, _NJ, carry=(zeros, zeros),
                                      unroll=8)(p1)

            means = []
            invs = []
            for tp in range(_TG):
                mean = _splat(jnp.sum(s[tp]), jnp.float32) * inv_h
                var = (_splat(jnp.sum(q[tp]), jnp.float32) * inv_h
                       - mean * mean)
                means.append(mean)
                invs.append(_rsqrt16(var + _EPS))

            @plsc.parallel_loop(0, _NJ, unroll=8)
            def p2(j):
                jo = j * _L
                gj = g_v[pl.ds(jo, _L)]
                bj = b_v[pl.ds(jo, _L)]
                xs = [wb[buf, t0 + tp, pl.ds(jo, _L)] for tp in range(_TG)]
                for tp in range(_TG):
                    wb[buf, t0 + tp, pl.ds(jo, _L)] = (
                        (xs[tp] - means[tp]) * invs[tp] * gj + bj)
            return 0

        lax.fori_loop(0, _CH // _TG, group, 0)

    issue_gather_w(0, 0)
    issue_gather_w(1, 1)
    wait_gather_w(0, 0)
    issue_gather_p(0, 0)

    def main(k4, _):
        for buf in range(_NBUF):
            k = k4 * _NBUF + buf
            nxt = (buf + 2) % _NBUF
            nx1 = (buf + 1) % _NBUF

            if buf >= 2:
                drain_out(nxt)
            else:
                @pl.when(k4 >= 1)
                def _():
                    drain_out(nxt)

            @pl.when(k + 2 < _NCHUNK)
            def _():
                issue_gather_w(k + 2, nxt)

            @pl.when(k + 1 < _NCHUNK)
            def _():
                wait_gather_w(k + 1, nx1)
                issue_gather_p(k + 1, nx1)

            wait_gather_p(k, buf)
            compute_chunk(buf)
            pltpu.async_copy(wb.at[buf], out_hbm.at[pl.ds(base + k * _CH, _CH)],
                             sems.at[2 * _NBUF + buf])
        return 0

    lax.fori_loop(0, _NCHUNK // _NBUF, main, 0)
    drain_out((_NCHUNK - 2) % _NBUF)
    drain_out((_NCHUNK - 1) % _NBUF)


@jax.jit
def kernel(input_ids, word_table, pos_table, tt_table, gamma, beta):
    mesh = plsc.VectorSubcoreMesh(core_axis_name="c", subcore_axis_name="s",
                                  num_cores=_NC, num_subcores=_NS)
    run = pl.kernel(
        _emb_body,
        out_type=jax.ShapeDtypeStruct((_B * _S, _H), jnp.float32),
        mesh=mesh,
        scratch_types=[
            pltpu.VMEM((_TOK_W,), jnp.int32),
            pltpu.VMEM((_TOK_W,), jnp.int32),
            pltpu.VMEM((_NBUF, _CH, _H), jnp.float32),
            pltpu.VMEM((_H,), jnp.float32),
            pltpu.VMEM((_H,), jnp.float32),
            pltpu.SemaphoreType.DMA((3 * _NBUF,)),
        ],
        compiler_params=pltpu.CompilerParams(use_tc_tiling_on_sc=False,
                                             needs_layout_passes=False),
        name="clap_text_embeddings_sc",
    )
    ptt = pos_table + tt_table[0][None, :]
    out = run(input_ids.astype(jnp.int32).reshape(-1), word_table, ptt,
              gamma, beta)
    return out.reshape(_B, _S, _H)


# speedup vs baseline: 1.0960x; 1.0960x over previous
import functools
import jax
import jax.numpy as jnp
from jax import lax
from jax.experimental import pallas as pl
from jax.experimental.pallas import tpu as pltpu
from jax.experimental.pallas import tpu_sc as plsc

_VOCAB = 50265
_H = 768
_PAD = 1
_EPS = 1e-12
_B, _S = 1024, 200

_L = 16
_NC, _NS = 2, 16
_NW = _NC * _NS
_TOK_W = _B * _S // _NW
_CH = 32
_NCHUNK = _TOK_W // _CH
_NBUF = 4
_TG = 8
_NJ = _H // _L


def _rsqrt16(v):
    i = plsc.bitcast(v, jnp.int32)
    i = jnp.int32(0x5F3759DF) - lax.shift_right_arithmetic(i, 1)
    y = plsc.bitcast(i, jnp.float32)
    for _ in range(3):
        y = y * (1.5 - 0.5 * v * y * y)
    return y


def _splat(s, dtype):
    return lax.broadcast_in_dim(s, (_L,), ()).astype(dtype)


def _emb_body(ids_hbm, word_hbm, pos_hbm, g_hbm, beta_hbm, out_hbm,
              ids_v, pos_v, wb, g_v, b_v, sems):
    wid = lax.axis_index("s") * _NC + lax.axis_index("c")
    base = wid * _TOK_W

    pltpu.sync_copy(g_hbm, g_v)
    pltpu.sync_copy(beta_hbm, b_v)
    pltpu.sync_copy(ids_hbm.at[pl.ds(base, _TOK_W)], ids_v)

    lane = lax.iota(jnp.int32, _L)
    low8 = (lane < 8).astype(jnp.int32)

    def std_chunk(off, carry):
        v = ids_v[pl.ds(off, _L)]
        m = (v != _PAD).astype(jnp.int32)
        pos = (lax.cumsum(m, axis=0) + carry) * m + _PAD
        pos_v[pl.ds(off, _L)] = pos
        return carry + _splat(jnp.sum(m), jnp.int32)

    def row_pair(p, _):
        pbase = p * (2 * _S)

        def chA(j, carry):
            return std_chunk(pbase + j * _L, carry)

        carry = lax.fori_loop(0, 12, chA, jnp.zeros((_L,), jnp.int32))

        off = pbase + 12 * _L
        v = ids_v[pl.ds(off, _L)]
        m = (v != _PAD).astype(jnp.int32)
        mlo = m * low8
        mhi = m - mlo
        poslo = (lax.cumsum(mlo, axis=0) + carry) * m + _PAD
        poshi = lax.cumsum(mhi, axis=0) * m + _PAD
        pos_v[pl.ds(off, _L)] = jnp.where(lane < 8, poslo, poshi)
        carry = _splat(jnp.sum(mhi), jnp.int32)

        def chB(j, carry):
            return std_chunk(pbase + 13 * _L + j * _L, carry)

        lax.fori_loop(0, 12, chB, carry)
        return 0

    lax.fori_loop(0, _TOK_W // (2 * _S), row_pair, 0)

    inv_h = 1.0 / _H

    def issue_gather_w(k, buf):
        idx_w = ids_v.at[pl.ds(k * _CH, _CH)]
        pltpu.async_copy(word_hbm.at[idx_w], wb.at[buf], sems.at[buf])

    def wait_gather_w(k, buf):
        pltpu.make_async_copy(word_hbm.at[ids_v.at[pl.ds(k * _CH, _CH)]],
                              wb.at[buf], sems.at[buf]).wait()

    def issue_gather_p(k, buf):
        idx_p = pos_v.at[pl.ds(k * _CH, _CH)]
        pltpu.async_copy(pos_hbm.at[idx_p], wb.at[buf], sems.at[_NBUF + buf],
                         add=True)

    def wait_gather_p(k, buf):
        pltpu.make_async_copy(pos_hbm.at[pos_v.at[pl.ds(k * _CH, _CH)]],
                              wb.at[buf], sems.at[_NBUF + buf]).wait()

    def drain_out(buf):
        pltpu.make_async_copy(wb.at[buf], out_hbm.at[pl.ds(base, _CH)],
                              sems.at[2 * _NBUF + buf]).wait()

    def compute_chunk(buf):
        def group(gidx, _):
            t0 = gidx * _TG

            def p1(j, acc):
                s, q = acc
                jo = j * _L
                xs = [wb[buf, t0 + tp, pl.ds(jo, _L)] for tp in range(_TG)]
                s = tuple(s[tp] + xs[tp] for tp in range(_TG))
                q = tuple(q[tp] + xs[tp] * xs[tp] for tp in range(_TG))
                return (s, q)

            zeros = tuple(jnp.zeros((_L,), jnp.float32) for _ in range(_TG))
            s, q = plsc.parallel_loop(0, _NJ, carry=(zeros, zeros),
                                      unroll=4)(p1)

            means = []
            invs = []
            for tp in range(_TG):
                mean = _splat(jnp.sum(s[tp]), jnp.float32) * inv_h
                var = (_splat(jnp.sum(q[tp]), jnp.float32) * inv_h
                       - mean * mean)
                means.append(mean)
                invs.append(_rsqrt16(var + _EPS))

            @plsc.parallel_loop(0, _NJ, unroll=4)
            def p2(j):
                jo = j * _L
                gj = g_v[pl.ds(jo, _L)]
                bj = b_v[pl.ds(jo, _L)]
                xs = [wb[buf, t0 + tp, pl.ds(jo, _L)] for tp in range(_TG)]
                for tp in range(_TG):
                    wb[buf, t0 + tp, pl.ds(jo, _L)] = (
                        (xs[tp] - means[tp]) * invs[tp] * gj + bj)
            return 0

        lax.fori_loop(0, _CH // _TG, group, 0)

    issue_gather_w(0, 0)
    issue_gather_w(1, 1)
    wait_gather_w(0, 0)
    issue_gather_p(0, 0)

    def main(k4, _):
        for buf in range(_NBUF):
            k = k4 * _NBUF + buf
            nxt = (buf + 2) % _NBUF
            nx1 = (buf + 1) % _NBUF

            if buf >= 2:
                drain_out(nxt)
            else:
                @pl.when(k4 >= 1)
                def _():
                    drain_out(nxt)

            @pl.when(k + 2 < _NCHUNK)
            def _():
                issue_gather_w(k + 2, nxt)

            @pl.when(k + 1 < _NCHUNK)
            def _():
                wait_gather_w(k + 1, nx1)
                issue_gather_p(k + 1, nx1)

            wait_gather_p(k, buf)
            compute_chunk(buf)
            pltpu.async_copy(wb.at[buf], out_hbm.at[pl.ds(base + k * _CH, _CH)],
                             sems.at[2 * _NBUF + buf])
        return 0

    lax.fori_loop(0, _NCHUNK // _NBUF, main, 0)
    drain_out((_NCHUNK - 2) % _NBUF)
    drain_out((_NCHUNK - 1) % _NBUF)


@jax.jit
def kernel(input_ids, word_table, pos_table, tt_table, gamma, beta):
    mesh = plsc.VectorSubcoreMesh(core_axis_name="c", subcore_axis_name="s",
                                  num_cores=_NC, num_subcores=_NS)
    run = pl.kernel(
        _emb_body,
        out_type=jax.ShapeDtypeStruct((_B * _S, _H), jnp.float32),
        mesh=mesh,
        scratch_types=[
            pltpu.VMEM((_TOK_W,), jnp.int32),
            pltpu.VMEM((_TOK_W,), jnp.int32),
            pltpu.VMEM((_NBUF, _CH, _H), jnp.float32),
            pltpu.VMEM((_H,), jnp.float32),
            pltpu.VMEM((_H,), jnp.float32),
            pltpu.SemaphoreType.DMA((3 * _NBUF,)),
        ],
        compiler_params=pltpu.CompilerParams(use_tc_tiling_on_sc=False,
                                             needs_layout_passes=False),
        name="clap_text_embeddings_sc",
    )
    ptt = pos_table + tt_table[0][None, :]
    out = run(input_ids.astype(jnp.int32).reshape(-1), word_table, ptt,
              gamma, beta)
    return out.reshape(_B, _S, _H)
